# baseline (speedup 1.0000x reference)
import functools

import jax
import jax.numpy as jnp
from jax import lax
from jax.experimental import pallas as pl
from jax.experimental.pallas import tpu as pltpu
from jax.experimental.pallas import tpu_sc as plsc

N = 10000
E = 320000
D = 128
DE = 16

NP = 10240
NW = 32
K = 128
CH = 80
HCH = CH // 2
EPW = CH * K
EP = NW * EPW
RPT = NP // 16
EAW = 128

_mesh = plsc.VectorSubcoreMesh(core_axis_name="c", subcore_axis_name="s")


def _sc_segsum_h(h_hbm, src_hbm, dst_hbm, z_hbm, out_hbm,
                 src_v, dst_v, rows_v, acc_sh, sem0, sem1):
    c = lax.axis_index("c")
    s = lax.axis_index("s")
    wid = s * 2 + c

    pltpu.sync_copy(z_hbm, acc_sh.at[pl.ds(s * RPT, RPT)])

    pltpu.sync_copy(src_hbm.at[wid], src_v)
    plsc.subcore_barrier()

    def gather(g, buf, sem):
        return pltpu.async_copy(
            h_hbm.at[src_v.at[pl.ds(g * K, K)]], rows_v.at[buf], sem)

    for half in range(2):
        base = half * HCH
        pltpu.sync_copy(dst_hbm.at[wid, half], dst_v)
        gather(base, 0, sem0)
        gather(base + 1, 1, sem1)

        def body(i, _):
            g = base + 2 * i
            pltpu.make_async_copy(
                h_hbm.at[src_v.at[pl.ds(g * K, K)]], rows_v.at[0], sem0).wait()
            pltpu.sync_copy(rows_v.at[0], acc_sh.at[dst_v.at[2 * i]], add=True)

            @pl.when(2 * i + 2 < HCH)
            def _():
                gather(g + 2, 0, sem0)

            pltpu.make_async_copy(
                h_hbm.at[src_v.at[pl.ds((g + 1) * K, K)]], rows_v.at[1], sem1).wait()
            pltpu.sync_copy(rows_v.at[1], acc_sh.at[dst_v.at[2 * i + 1]], add=True)

            @pl.when(2 * i + 3 < HCH)
            def _():
                gather(g + 3, 1, sem1)

            return _

        lax.fori_loop(0, HCH // 2, body, None)

    plsc.subcore_barrier()
    pltpu.sync_copy(acc_sh.at[pl.ds(s * RPT, RPT)],
                    out_hbm.at[c, pl.ds(s * RPT, RPT)])


_segsum_h = functools.partial(
    pl.kernel,
    out_type=jax.ShapeDtypeStruct((2, NP, D), jnp.float32),
    mesh=_mesh,
    scratch_types=[
        pltpu.VMEM((EPW,), jnp.int32),
        pltpu.VMEM((HCH, K), jnp.int32),
        pltpu.VMEM((2, K, D), jnp.float32),
        pltpu.VMEM_SHARED((NP, D), jnp.float32),
        pltpu.SemaphoreType.DMA,
        pltpu.SemaphoreType.DMA,
    ],
)(_sc_segsum_h)


def _sc_segsum_ea(eap_hbm, dst_hbm, z_hbm, out_hbm,
                  dst_v, pk_v, rows_v, acc_sh, sem0, sem1):
    c = lax.axis_index("c")
    s = lax.axis_index("s")
    wid = s * 2 + c

    pltpu.sync_copy(z_hbm, acc_sh.at[pl.ds(s * RPT, RPT)])
    one = jnp.full((16,), 1.0, jnp.float32)
    zv = jnp.zeros((16,), jnp.float32)
    onehot = jnp.where(lax.iota(jnp.int32, 16) == 0, one, zv)
    for buf in range(2):
        for j in range(K):
            rows_v[buf, j, pl.ds(DE, 16)] = onehot
            for k2 in range(2, 8):
                rows_v[buf, j, pl.ds(k2 * 16, 16)] = zv

    plsc.subcore_barrier()

    def ld(g, buf, sem):
        return pltpu.async_copy(
            eap_hbm.at[pl.ds(wid * (EPW // 8) + g * (K // 8), K // 8)],
            pk_v.at[buf], sem)

    def unpack(buf):
        for pr in range(K // 8):
            for l in range(8):
                rows_v[buf, pr * 8 + l, pl.ds(0, DE)] = \
                    pk_v[buf, pr, pl.ds(l * DE, DE)]

    for half in range(2):
        base = half * HCH
        pltpu.sync_copy(dst_hbm.at[wid, half], dst_v)
        ld(base, 0, sem0)
        ld(base + 1, 1, sem1)

        def body(i, _):
            g = base + 2 * i
            pltpu.make_async_copy(
                eap_hbm.at[pl.ds(wid * (EPW // 8) + g * (K // 8), K // 8)],
                pk_v.at[0], sem0).wait()
            unpack(0)
            pltpu.sync_copy(rows_v.at[0], acc_sh.at[dst_v.at[2 * i]], add=True)

            @pl.when(2 * i + 2 < HCH)
            def _():
                ld(g + 2, 0, sem0)

            pltpu.make_async_copy(
                eap_hbm.at[pl.ds(wid * (EPW // 8) + (g + 1) * (K // 8), K // 8)],
                pk_v.at[1], sem1).wait()
            unpack(1)
            pltpu.sync_copy(rows_v.at[1], acc_sh.at[dst_v.at[2 * i + 1]], add=True)

            @pl.when(2 * i + 3 < HCH)
            def _():
                ld(g + 3, 1, sem1)

            return _

        lax.fori_loop(0, HCH // 2, body, None)

    plsc.subcore_barrier()
    pltpu.sync_copy(acc_sh.at[pl.ds(s * RPT, RPT)],
                    out_hbm.at[c, pl.ds(s * RPT, RPT)])


_segsum_ea = functools.partial(
    pl.kernel,
    out_type=jax.ShapeDtypeStruct((2, NP, EAW), jnp.float32),
    mesh=_mesh,
    scratch_types=[
        pltpu.VMEM((HCH, K), jnp.int32),
        pltpu.VMEM((2, K // 8, EAW), jnp.float32),
        pltpu.VMEM((2, K, EAW), jnp.float32),
        pltpu.VMEM_SHARED((NP, EAW), jnp.float32),
        pltpu.SemaphoreType.DMA,
        pltpu.SemaphoreType.DMA,
    ],
)(_sc_segsum_ea)


def _epi_body(relu, acc_ref, h_ref, ea_ref, ew_ref, eb_ref, w_ref, b_ref, o_ref):
    ea2 = ea_ref[0] + ea_ref[1]
    cnt = ea2[:, DE:DE + 1] + 1.0
    edge_term = lax.dot_general(ea2[:, :DE], ew_ref[...],
                                (((1,), (1,)), ((), ())),
                                preferred_element_type=jnp.float32)
    sacc = acc_ref[0] + acc_ref[1] + h_ref[...] + edge_term + cnt * eb_ref[...]
    aggr = sacc / cnt
    out = jnp.dot(aggr, w_ref[...], preferred_element_type=jnp.float32) + b_ref[...]
    o_ref[...] = jnp.maximum(out, 0.0) if relu else out


def _epilogue(acc, h, ea, ew, eb, w, b, relu):
    blk = 1024
    grid = NP // blk
    return pl.pallas_call(
        functools.partial(_epi_body, relu),
        grid=(grid,),
        in_specs=[
            pl.BlockSpec((2, blk, D), lambda i: (0, i, 0)),
            pl.BlockSpec((blk, D), lambda i: (i, 0)),
            pl.BlockSpec((2, blk, EAW), lambda i: (0, i, 0)),
            pl.BlockSpec((D, DE), lambda i: (0, 0)),
            pl.BlockSpec((1, D), lambda i: (0, 0)),
            pl.BlockSpec((D, D), lambda i: (0, 0)),
            pl.BlockSpec((1, D), lambda i: (0, 0)),
        ],
        out_specs=pl.BlockSpec((blk, D), lambda i: (i, 0)),
        out_shape=jax.ShapeDtypeStruct((NP, D), jnp.float32),
    )(acc, h, ea, ew, eb.reshape(1, D), w, b.reshape(1, D))


def kernel(x, edge_index, edge_attr,
           w1, b1, ew1, eb1, w2, b2, ew2, eb2, w3, b3, ew3, eb3,
           w4, b4, ew4, eb4, w5, b5, ew5, eb5):
    ei = edge_index.astype(jnp.int32)
    pad = EP - E
    pad_src = (jnp.arange(pad, dtype=jnp.int32) * 131) % N
    pad_dst = N + (jnp.arange(pad, dtype=jnp.int32) % (NP - N))
    src2 = jnp.concatenate([ei[0], pad_src]).reshape(NW, EPW)
    dst4 = jnp.concatenate([ei[1], pad_dst]).reshape(NW, 2, HCH, K)

    eap = jnp.concatenate(
        [edge_attr, jnp.zeros((pad, DE), jnp.float32)], axis=0
    ).reshape(EP // 8, 8 * DE)

    h = jnp.concatenate([x, jnp.zeros((NP - N, D), jnp.float32)], axis=0)
    z_h = jnp.zeros((RPT, D), jnp.float32)

    ea_acc = _segsum_ea(eap, dst4, z_h)

    params = [(w1, b1, ew1, eb1), (w2, b2, ew2, eb2), (w3, b3, ew3, eb3),
              (w4, b4, ew4, eb4), (w5, b5, ew5, eb5)]
    for i, (w, b, ew, eb) in enumerate(params):
        acc = _segsum_h(h, src2, dst4, z_h)
        h = _epilogue(acc, h, ea_acc, ew, eb, w, b, relu=i < 4)
    return h[:N]


# speedup vs baseline: 1.0132x; 1.0132x over previous
import functools

import jax
import jax.numpy as jnp
from jax import lax
from jax.experimental import pallas as pl
from jax.experimental.pallas import tpu as pltpu
from jax.experimental.pallas import tpu_sc as plsc

N = 10000
E = 320000
D = 128
DE = 16

NP = 10240
NW = 32
K = 128
CH = 80
HCH = CH // 2
EPW = CH * K
EP = NW * EPW
RPT = NP // 16
EAW = 128

_mesh = plsc.VectorSubcoreMesh(core_axis_name="c", subcore_axis_name="s")


def _sc_segsum_h(h_hbm, src_hbm, dst_hbm, z_hbm, out_hbm,
                 src_v, dst_v, rows_v, acc_sh, sem0, sem1):
    c = lax.axis_index("c")
    s = lax.axis_index("s")
    wid = s * 2 + c

    @pl.when(c == 0)
    def _():
        pltpu.sync_copy(h_hbm.at[pl.ds(s * RPT, RPT)],
                        acc_sh.at[pl.ds(s * RPT, RPT)])

    @pl.when(c == 1)
    def _():
        pltpu.sync_copy(z_hbm, acc_sh.at[pl.ds(s * RPT, RPT)])

    pltpu.sync_copy(src_hbm.at[wid], src_v)
    plsc.subcore_barrier()

    def gather(g, buf, sem):
        return pltpu.async_copy(
            h_hbm.at[src_v.at[pl.ds(g * K, K)]], rows_v.at[buf], sem)

    for half in range(2):
        base = half * HCH
        pltpu.sync_copy(dst_hbm.at[wid, half], dst_v)
        gather(base, 0, sem0)
        gather(base + 1, 1, sem1)

        def body(i, _):
            g = base + 2 * i
            pltpu.make_async_copy(
                h_hbm.at[src_v.at[pl.ds(g * K, K)]], rows_v.at[0], sem0).wait()
            pltpu.sync_copy(rows_v.at[0], acc_sh.at[dst_v.at[2 * i]], add=True)

            @pl.when(2 * i + 2 < HCH)
            def _():
                gather(g + 2, 0, sem0)

            pltpu.make_async_copy(
                h_hbm.at[src_v.at[pl.ds((g + 1) * K, K)]], rows_v.at[1], sem1).wait()
            pltpu.sync_copy(rows_v.at[1], acc_sh.at[dst_v.at[2 * i + 1]], add=True)

            @pl.when(2 * i + 3 < HCH)
            def _():
                gather(g + 3, 1, sem1)

            return _

        lax.fori_loop(0, HCH // 2, body, None)

    plsc.subcore_barrier()
    pltpu.sync_copy(acc_sh.at[pl.ds(s * RPT, RPT)],
                    out_hbm.at[c, pl.ds(s * RPT, RPT)])


_segsum_h = functools.partial(
    pl.kernel,
    out_type=jax.ShapeDtypeStruct((2, NP, D), jnp.float32),
    mesh=_mesh,
    scratch_types=[
        pltpu.VMEM((EPW,), jnp.int32),
        pltpu.VMEM((HCH, K), jnp.int32),
        pltpu.VMEM((2, K, D), jnp.float32),
        pltpu.VMEM_SHARED((NP, D), jnp.float32),
        pltpu.SemaphoreType.DMA,
        pltpu.SemaphoreType.DMA,
    ],
)(_sc_segsum_h)


def _sc_segsum_ea(eap_hbm, dst_hbm, z_hbm, out_hbm,
                  dst_v, pk_v, rows_v, acc_sh, sem0, sem1, semS):
    c = lax.axis_index("c")
    s = lax.axis_index("s")
    wid = s * 2 + c

    pltpu.sync_copy(z_hbm, acc_sh.at[pl.ds(s * RPT, RPT)])
    one = jnp.full((16,), 1.0, jnp.float32)
    zv = jnp.zeros((16,), jnp.float32)
    onehot = jnp.where(lax.iota(jnp.int32, 16) == 0, one, zv)
    for buf in range(2):
        for j in range(K):
            rows_v[buf, j, pl.ds(DE, 16)] = onehot
            for k2 in range(2, 8):
                rows_v[buf, j, pl.ds(k2 * 16, 16)] = zv

    plsc.subcore_barrier()

    def ld(g, buf, sem):
        return pltpu.async_copy(
            eap_hbm.at[pl.ds(wid * (EPW // 8) + g * (K // 8), K // 8)],
            pk_v.at[buf], sem)

    def unpack(buf):
        for pr in range(K // 8):
            for l in range(8):
                rows_v[buf, pr * 8 + l, pl.ds(0, DE)] = \
                    pk_v[buf, pr, pl.ds(l * DE, DE)]

    for half in range(2):
        base = half * HCH
        pltpu.sync_copy(dst_hbm.at[wid, half], dst_v)
        ld(base, 0, sem0)
        ld(base + 1, 1, sem1)

        def body(i, _):
            g = base + 2 * i
            pltpu.make_async_copy(
                eap_hbm.at[pl.ds(wid * (EPW // 8) + g * (K // 8), K // 8)],
                pk_v.at[0], sem0).wait()
            unpack(0)
            pltpu.async_copy(rows_v.at[0], acc_sh.at[dst_v.at[2 * i]], semS,
                             add=True)

            pltpu.make_async_copy(
                eap_hbm.at[pl.ds(wid * (EPW // 8) + (g + 1) * (K // 8), K // 8)],
                pk_v.at[1], sem1).wait()
            unpack(1)

            @pl.when(2 * i + 2 < HCH)
            def _():
                ld(g + 2, 0, sem0)

            pltpu.make_async_copy(rows_v.at[0], acc_sh.at[dst_v.at[2 * i]],
                                  semS).wait()
            pltpu.sync_copy(rows_v.at[1], acc_sh.at[dst_v.at[2 * i + 1]], add=True)

            @pl.when(2 * i + 3 < HCH)
            def _():
                ld(g + 3, 1, sem1)

            return _

        lax.fori_loop(0, HCH // 2, body, None)

    plsc.subcore_barrier()
    pltpu.sync_copy(acc_sh.at[pl.ds(s * RPT, RPT)],
                    out_hbm.at[c, pl.ds(s * RPT, RPT)])


_segsum_ea = functools.partial(
    pl.kernel,
    out_type=jax.ShapeDtypeStruct((2, NP, EAW), jnp.float32),
    mesh=_mesh,
    scratch_types=[
        pltpu.VMEM((HCH, K), jnp.int32),
        pltpu.VMEM((2, K // 8, EAW), jnp.float32),
        pltpu.VMEM((2, K, EAW), jnp.float32),
        pltpu.VMEM_SHARED((NP, EAW), jnp.float32),
        pltpu.SemaphoreType.DMA,
        pltpu.SemaphoreType.DMA,
        pltpu.SemaphoreType.DMA,
    ],
)(_sc_segsum_ea)


def _epi_body(relu, acc_ref, ea_ref, ew_ref, eb_ref, w_ref, b_ref, o_ref):
    ea2 = ea_ref[0] + ea_ref[1]
    cnt = ea2[:, DE:DE + 1] + 1.0
    edge_term = lax.dot_general(ea2[:, :DE], ew_ref[...],
                                (((1,), (1,)), ((), ())),
                                preferred_element_type=jnp.float32)
    sacc = acc_ref[0] + acc_ref[1] + edge_term + cnt * eb_ref[...]
    aggr = sacc / cnt
    out = jnp.dot(aggr, w_ref[...], preferred_element_type=jnp.float32) + b_ref[...]
    o_ref[...] = jnp.maximum(out, 0.0) if relu else out


def _epilogue(acc, ea, ew, eb, w, b, relu):
    blk = 1024
    grid = NP // blk
    return pl.pallas_call(
        functools.partial(_epi_body, relu),
        grid=(grid,),
        in_specs=[
            pl.BlockSpec((2, blk, D), lambda i: (0, i, 0)),
            pl.BlockSpec((2, blk, EAW), lambda i: (0, i, 0)),
            pl.BlockSpec((D, DE), lambda i: (0, 0)),
            pl.BlockSpec((1, D), lambda i: (0, 0)),
            pl.BlockSpec((D, D), lambda i: (0, 0)),
            pl.BlockSpec((1, D), lambda i: (0, 0)),
        ],
        out_specs=pl.BlockSpec((blk, D), lambda i: (i, 0)),
        out_shape=jax.ShapeDtypeStruct((NP, D), jnp.float32),
    )(acc, ea, ew, eb.reshape(1, D), w, b.reshape(1, D))


def kernel(x, edge_index, edge_attr,
           w1, b1, ew1, eb1, w2, b2, ew2, eb2, w3, b3, ew3, eb3,
           w4, b4, ew4, eb4, w5, b5, ew5, eb5):
    ei = edge_index.astype(jnp.int32)
    pad = EP - E
    pad_src = (jnp.arange(pad, dtype=jnp.int32) * 131) % N
    pad_dst = N + (jnp.arange(pad, dtype=jnp.int32) % (NP - N))
    src2 = jnp.concatenate([ei[0], pad_src]).reshape(NW, EPW)
    dst4 = jnp.concatenate([ei[1], pad_dst]).reshape(NW, 2, HCH, K)

    eap = jnp.concatenate(
        [edge_attr, jnp.zeros((pad, DE), jnp.float32)], axis=0
    ).reshape(EP // 8, 8 * DE)

    h = jnp.concatenate([x, jnp.zeros((NP - N, D), jnp.float32)], axis=0)
    z_h = jnp.zeros((RPT, D), jnp.float32)

    ea_acc = _segsum_ea(eap, dst4, z_h)

    params = [(w1, b1, ew1, eb1), (w2, b2, ew2, eb2), (w3, b3, ew3, eb3),
              (w4, b4, ew4, eb4), (w5, b5, ew5, eb5)]
    for i, (w, b, ew, eb) in enumerate(params):
        acc = _segsum_h(h, src2, dst4, z_h)
        h = _epilogue(acc, ea_acc, ew, eb, w, b, relu=i < 4)
    return h[:N]


# speedup vs baseline: 1.0264x; 1.0130x over previous
import functools

import jax
import jax.numpy as jnp
from jax import lax
from jax.experimental import pallas as pl
from jax.experimental.pallas import tpu as pltpu
from jax.experimental.pallas import tpu_sc as plsc

N = 10000
E = 320000
D = 128
DE = 16

NP = 10240
NW = 32
K = 128
CH = 80
HCH = CH // 2
EPW = CH * K
EP = NW * EPW
RPT = NP // 16
EAW = 128

_mesh = plsc.VectorSubcoreMesh(core_axis_name="c", subcore_axis_name="s")


def _sc_segsum_h(h_hbm, src_hbm, dst_hbm, z_hbm, out_hbm,
                 src_v, dst_v, rows_v, acc_sh, sem0, sem1):
    c = lax.axis_index("c")
    s = lax.axis_index("s")
    wid = s * 2 + c

    @pl.when(c == 0)
    def _():
        pltpu.sync_copy(h_hbm.at[pl.ds(s * RPT, RPT)],
                        acc_sh.at[pl.ds(s * RPT, RPT)])

    @pl.when(c == 1)
    def _():
        pltpu.sync_copy(z_hbm, acc_sh.at[pl.ds(s * RPT, RPT)])

    pltpu.sync_copy(src_hbm.at[wid], src_v)
    plsc.subcore_barrier()

    def gather(g, buf, sem):
        return pltpu.async_copy(
            h_hbm.at[src_v.at[pl.ds(g * K, K)]], rows_v.at[buf], sem)

    for half in range(2):
        base = half * HCH
        pltpu.sync_copy(dst_hbm.at[wid, half], dst_v)
        gather(base, 0, sem0)
        gather(base + 1, 1, sem1)

        def body(i, _):
            g = base + 2 * i
            pltpu.make_async_copy(
                h_hbm.at[src_v.at[pl.ds(g * K, K)]], rows_v.at[0], sem0).wait()
            pltpu.sync_copy(rows_v.at[0], acc_sh.at[dst_v.at[2 * i]], add=True)

            @pl.when(2 * i + 2 < HCH)
            def _():
                gather(g + 2, 0, sem0)

            pltpu.make_async_copy(
                h_hbm.at[src_v.at[pl.ds((g + 1) * K, K)]], rows_v.at[1], sem1).wait()
            pltpu.sync_copy(rows_v.at[1], acc_sh.at[dst_v.at[2 * i + 1]], add=True)

            @pl.when(2 * i + 3 < HCH)
            def _():
                gather(g + 3, 1, sem1)

            return _

        lax.fori_loop(0, HCH // 2, body, None)

    plsc.subcore_barrier()
    pltpu.sync_copy(acc_sh.at[pl.ds(s * RPT, RPT)],
                    out_hbm.at[c, pl.ds(s * RPT, RPT)])


_segsum_h = functools.partial(
    pl.kernel,
    out_type=jax.ShapeDtypeStruct((2, NP, D), jnp.float32),
    mesh=_mesh,
    scratch_types=[
        pltpu.VMEM((EPW,), jnp.int32),
        pltpu.VMEM((HCH, K), jnp.int32),
        pltpu.VMEM((2, K, D), jnp.float32),
        pltpu.VMEM_SHARED((NP, D), jnp.float32),
        pltpu.SemaphoreType.DMA,
        pltpu.SemaphoreType.DMA,
    ],
)(_sc_segsum_h)


def _sc_segsum_ea(eap_hbm, dst_hbm, z_hbm, out_hbm,
                  dst_v, pk_v, rows_v, acc_sh, sem0, sem1, semS):
    c = lax.axis_index("c")
    s = lax.axis_index("s")
    wid = s * 2 + c

    pltpu.sync_copy(z_hbm, acc_sh.at[pl.ds(s * RPT, RPT)])
    one = jnp.full((16,), 1.0, jnp.float32)
    zv = jnp.zeros((16,), jnp.float32)
    onehot = jnp.where(lax.iota(jnp.int32, 16) == 0, one, zv)
    for buf in range(2):
        for j in range(K):
            rows_v[buf, j, pl.ds(DE, 16)] = onehot
            for k2 in range(2, 8):
                rows_v[buf, j, pl.ds(k2 * 16, 16)] = zv

    plsc.subcore_barrier()

    def ld(g, buf, sem):
        return pltpu.async_copy(
            eap_hbm.at[pl.ds(wid * (EPW // 8) + g * (K // 8), K // 8)],
            pk_v.at[buf], sem)

    def unpack(buf):
        for pr in range(K // 8):
            for l in range(8):
                rows_v[buf, pr * 8 + l, pl.ds(0, DE)] = \
                    pk_v[buf, pr, pl.ds(l * DE, DE)]

    for half in range(2):
        base = half * HCH
        pltpu.sync_copy(dst_hbm.at[wid, half], dst_v)
        ld(base, 0, sem0)
        ld(base + 1, 1, sem1)

        def body(i, _):
            g = base + 2 * i
            pltpu.make_async_copy(
                eap_hbm.at[pl.ds(wid * (EPW // 8) + g * (K // 8), K // 8)],
                pk_v.at[0], sem0).wait()
            unpack(0)
            pltpu.async_copy(rows_v.at[0], acc_sh.at[dst_v.at[2 * i]], semS,
                             add=True)

            pltpu.make_async_copy(
                eap_hbm.at[pl.ds(wid * (EPW // 8) + (g + 1) * (K // 8), K // 8)],
                pk_v.at[1], sem1).wait()
            unpack(1)

            @pl.when(2 * i + 2 < HCH)
            def _():
                ld(g + 2, 0, sem0)

            pltpu.make_async_copy(rows_v.at[0], acc_sh.at[dst_v.at[2 * i]],
                                  semS).wait()
            pltpu.sync_copy(rows_v.at[1], acc_sh.at[dst_v.at[2 * i + 1]], add=True)

            @pl.when(2 * i + 3 < HCH)
            def _():
                ld(g + 3, 1, sem1)

            return _

        lax.fori_loop(0, HCH // 2, body, None)

    plsc.subcore_barrier()
    pltpu.sync_copy(acc_sh.at[pl.ds(s * RPT, RPT)],
                    out_hbm.at[c, pl.ds(s * RPT, RPT)])


_segsum_ea = functools.partial(
    pl.kernel,
    out_type=jax.ShapeDtypeStruct((2, NP, EAW), jnp.float32),
    mesh=_mesh,
    scratch_types=[
        pltpu.VMEM((HCH, K), jnp.int32),
        pltpu.VMEM((2, K // 8, EAW), jnp.float32),
        pltpu.VMEM((2, K, EAW), jnp.float32),
        pltpu.VMEM_SHARED((NP, EAW), jnp.float32),
        pltpu.SemaphoreType.DMA,
        pltpu.SemaphoreType.DMA,
        pltpu.SemaphoreType.DMA,
    ],
)(_sc_segsum_ea)


def _epi_body(relu, acc_ref, ea_ref, ew_ref, eb_ref, w_ref, b_ref, o_ref):
    ea2 = ea_ref[0] + ea_ref[1]
    cnt = ea2[:, DE:DE + 1] + 1.0
    edge_term = lax.dot_general(ea2[:, :DE], ew_ref[...],
                                (((1,), (1,)), ((), ())),
                                preferred_element_type=jnp.float32)
    sacc = acc_ref[0] + acc_ref[1] + edge_term + cnt * eb_ref[...]
    aggr = sacc / cnt
    out = jnp.dot(aggr, w_ref[...], preferred_element_type=jnp.float32) + b_ref[...]
    o_ref[...] = jnp.maximum(out, 0.0) if relu else out


def _epilogue(acc, ea, ew, eb, w, b, relu):
    blk = 2048
    grid = NP // blk
    return pl.pallas_call(
        functools.partial(_epi_body, relu),
        grid=(grid,),
        in_specs=[
            pl.BlockSpec((2, blk, D), lambda i: (0, i, 0)),
            pl.BlockSpec((2, blk, EAW), lambda i: (0, i, 0)),
            pl.BlockSpec((D, DE), lambda i: (0, 0)),
            pl.BlockSpec((1, D), lambda i: (0, 0)),
            pl.BlockSpec((D, D), lambda i: (0, 0)),
            pl.BlockSpec((1, D), lambda i: (0, 0)),
        ],
        out_specs=pl.BlockSpec((blk, D), lambda i: (i, 0)),
        out_shape=jax.ShapeDtypeStruct((NP, D), jnp.float32),
    )(acc, ea, ew, eb.reshape(1, D), w, b.reshape(1, D))


def kernel(x, edge_index, edge_attr,
           w1, b1, ew1, eb1, w2, b2, ew2, eb2, w3, b3, ew3, eb3,
           w4, b4, ew4, eb4, w5, b5, ew5, eb5):
    ei = edge_index.astype(jnp.int32)
    pad = EP - E
    pad_src = (jnp.arange(pad, dtype=jnp.int32) * 131) % N
    pad_dst = N + (jnp.arange(pad, dtype=jnp.int32) % (NP - N))
    src2 = jnp.concatenate([ei[0], pad_src]).reshape(NW, EPW)
    dst4 = jnp.concatenate([ei[1], pad_dst]).reshape(NW, 2, HCH, K)

    eap = jnp.concatenate(
        [edge_attr, jnp.zeros((pad, DE), jnp.float32)], axis=0
    ).reshape(EP // 8, 8 * DE)

    h = jnp.concatenate([x, jnp.zeros((NP - N, D), jnp.float32)], axis=0)
    z_h = jnp.zeros((RPT, D), jnp.float32)

    ea_acc = _segsum_ea(eap, dst4, z_h)

    params = [(w1, b1, ew1, eb1), (w2, b2, ew2, eb2), (w3, b3, ew3, eb3),
              (w4, b4, ew4, eb4), (w5, b5, ew5, eb5)]
    for i, (w, b, ew, eb) in enumerate(params):
        acc = _segsum_h(h, src2, dst4, z_h)
        h = _epilogue(acc, ea_acc, ew, eb, w, b, relu=i < 4)
    return h[:N]
